# baseline (speedup 1.0000x reference)
import functools

import jax
import jax.numpy as jnp
from jax import lax
from jax.experimental import pallas as pl
from jax.experimental.pallas import tpu as pltpu
from jax.experimental.pallas import tpu_sc as plsc

F32 = jnp.float32
_NC, _NS = 2, 16
_NW = _NC * _NS
_CH = 128


def _celu(u):
    return jnp.where(u > 0, u, jnp.exp(jnp.minimum(u, 0.0)) - 1.0)


def _ln(y, g, b):
    mu = jnp.mean(y, axis=-1, keepdims=True)
    var = jnp.mean((y - mu) ** 2, axis=-1, keepdims=True)
    return (y - mu) * lax.rsqrt(var + 1e-5) * g + b


def _mlp2_ln_body(x_ref, w1_ref, b1_ref, w2_ref, b2_ref, g_ref, be_ref, o_ref):
    u = _celu(jnp.dot(x_ref[...], w1_ref[...], preferred_element_type=F32)
              + b1_ref[...])
    y = jnp.dot(u, w2_ref[...], preferred_element_type=F32) + b2_ref[...]
    o_ref[...] = _ln(y, g_ref[...], be_ref[...])


def _mlp2_ln(x, w1, b1, w2, b2, g, be, bm):
    n, kdim = x.shape
    grid = n // bm
    return pl.pallas_call(
        _mlp2_ln_body,
        grid=(grid,),
        in_specs=[
            pl.BlockSpec((bm, kdim), lambda i: (i, 0)),
            pl.BlockSpec((kdim, 128), lambda i: (0, 0)),
            pl.BlockSpec((1, 128), lambda i: (0, 0)),
            pl.BlockSpec((128, 128), lambda i: (0, 0)),
            pl.BlockSpec((1, 128), lambda i: (0, 0)),
            pl.BlockSpec((1, 128), lambda i: (0, 0)),
            pl.BlockSpec((1, 128), lambda i: (0, 0)),
        ],
        out_specs=pl.BlockSpec((bm, 128), lambda i: (i, 0)),
        out_shape=jax.ShapeDtypeStruct((n, 128), F32),
    )(x, w1, b1.reshape(1, 128), w2, b2.reshape(1, 128),
      g.reshape(1, 128), be.reshape(1, 128))


def _matmul_body(x_ref, w_ref, o_ref):
    o_ref[...] = jnp.dot(x_ref[...], w_ref[...], preferred_element_type=F32)


def _edge_pre(e_fwd, w1a, bm=1000):
    n = e_fwd.shape[0]
    return pl.pallas_call(
        _matmul_body,
        grid=(n // bm,),
        in_specs=[
            pl.BlockSpec((bm, 128), lambda i: (i, 0)),
            pl.BlockSpec((128, 128), lambda i: (0, 0)),
        ],
        out_specs=pl.BlockSpec((bm, 128), lambda i: (i, 0)),
        out_shape=jax.ShapeDtypeStruct((n, 128), F32),
    )(e_fwd, w1a)


def _tables_body(h_ref, w_ref, o_ref):
    o_ref[...] = jnp.dot(h_ref[...], w_ref[0], preferred_element_type=F32)


def _tables(h, w1b, w1c, bm=1000):
    n = h.shape[0]
    nb = n // bm
    wbc = jnp.stack([w1b, w1c])
    return pl.pallas_call(
        _tables_body,
        grid=(2 * nb,),
        in_specs=[
            pl.BlockSpec((bm, 128), lambda i: (i % nb, 0)),
            pl.BlockSpec((1, 128, 128), lambda i: (i // nb, 0, 0)),
        ],
        out_specs=pl.BlockSpec((bm, 128), lambda i: (i, 0)),
        out_shape=jax.ShapeDtypeStruct((2 * n, 128), F32),
    )(h, wbc)


def _edge_post_body(a_ref, gp_ref, gq_ref, e_ref, b1_ref, w2_ref, b2_ref,
                    g_ref, be_ref, msg_ref, enew_ref):
    u = _celu(a_ref[...] + gp_ref[0] + gq_ref[0] + b1_ref[...])
    m = _ln(jnp.dot(u, w2_ref[...], preferred_element_type=F32) + b2_ref[...],
            g_ref[...], be_ref[...])
    msg_ref[...] = m
    enew_ref[...] = e_ref[...] + m


def _edge_post(a, gfull, e_fwd, b1, w2, b2, g, be, bm=1000):
    n = a.shape[0]
    nb = n // bm
    return pl.pallas_call(
        _edge_post_body,
        grid=(nb,),
        in_specs=[
            pl.BlockSpec((bm, 128), lambda i: (i, 0)),
            pl.BlockSpec((1, bm, 128), lambda i: (0, i, 0)),
            pl.BlockSpec((1, bm, 128), lambda i: (1, i, 0)),
            pl.BlockSpec((bm, 128), lambda i: (i, 0)),
            pl.BlockSpec((1, 128), lambda i: (0, 0)),
            pl.BlockSpec((128, 128), lambda i: (0, 0)),
            pl.BlockSpec((1, 128), lambda i: (0, 0)),
            pl.BlockSpec((1, 128), lambda i: (0, 0)),
            pl.BlockSpec((1, 128), lambda i: (0, 0)),
        ],
        out_specs=[
            pl.BlockSpec((bm, 128), lambda i: (i, 0)),
            pl.BlockSpec((bm, 128), lambda i: (i, 0)),
        ],
        out_shape=[
            jax.ShapeDtypeStruct((n, 128), F32),
            jax.ShapeDtypeStruct((n, 128), F32),
        ],
    )(a, gfull, gfull, e_fwd, b1.reshape(1, 128), w2, b2.reshape(1, 128),
      g.reshape(1, 128), be.reshape(1, 128))


def _node_body(h_ref, p0_ref, p1_ref, v1a_ref, v1b_ref, c1_ref, v2_ref,
               c2_ref, g_ref, be_ref, o_ref):
    agg = p0_ref[0] - p1_ref[0]
    u = _celu(jnp.dot(h_ref[...], v1a_ref[...], preferred_element_type=F32)
              + jnp.dot(agg, v1b_ref[...], preferred_element_type=F32)
              + c1_ref[...])
    y = _ln(jnp.dot(u, v2_ref[...], preferred_element_type=F32) + c2_ref[...],
            g_ref[...], be_ref[...])
    o_ref[...] = h_ref[...] + y


def _node_update(h, partials, v1a, v1b, c1, v2, c2, g, be, bm=1000):
    n = h.shape[0]
    return pl.pallas_call(
        _node_body,
        grid=(n // bm,),
        in_specs=[
            pl.BlockSpec((bm, 128), lambda i: (i, 0)),
            pl.BlockSpec((1, bm, 128), lambda i: (0, i, 0)),
            pl.BlockSpec((1, bm, 128), lambda i: (1, i, 0)),
            pl.BlockSpec((128, 128), lambda i: (0, 0)),
            pl.BlockSpec((128, 128), lambda i: (0, 0)),
            pl.BlockSpec((1, 128), lambda i: (0, 0)),
            pl.BlockSpec((128, 128), lambda i: (0, 0)),
            pl.BlockSpec((1, 128), lambda i: (0, 0)),
            pl.BlockSpec((1, 128), lambda i: (0, 0)),
            pl.BlockSpec((1, 128), lambda i: (0, 0)),
        ],
        out_specs=pl.BlockSpec((bm, 128), lambda i: (i, 0)),
        out_shape=jax.ShapeDtypeStruct((n, 128), F32),
    )(h, partials, partials, v1a, v1b, c1.reshape(1, 128), v2,
      c2.reshape(1, 128), g.reshape(1, 128), be.reshape(1, 128))


def _ebwd_body(e0f_ref, e0b_ref, ef_ref, o_ref):
    o_ref[...] = e0b_ref[...] - (ef_ref[...] - e0f_ref[...])


def _ebwd(e0, ef, bm=1000):
    n = ef.shape[0]
    nb = n // bm
    return pl.pallas_call(
        _ebwd_body,
        grid=(nb,),
        in_specs=[
            pl.BlockSpec((bm, 128), lambda i: (i, 0)),
            pl.BlockSpec((bm, 128), lambda i: (i + nb, 0)),
            pl.BlockSpec((bm, 128), lambda i: (i, 0)),
        ],
        out_specs=pl.BlockSpec((bm, 128), lambda i: (i, 0)),
        out_shape=jax.ShapeDtypeStruct((n, 128), F32),
    )(e0, e0, ef)


def _final_body(h_ref, q0_ref, q1_ref, fg_ref, fb_ref, w1s_ref, b1s_ref,
                w2s_ref, b2v_ref, bcm_ref, o_ref):
    h = h_ref[...]
    inc = q0_ref[0] + q1_ref[0]
    s = jnp.sum(h, axis=-1, keepdims=True) + jnp.sum(inc, axis=-1, keepdims=True)
    mu = s / 256.0
    v = (jnp.sum((h - mu) ** 2, axis=-1, keepdims=True)
         + jnp.sum((inc - mu) ** 2, axis=-1, keepdims=True)) / 256.0
    rs = lax.rsqrt(v + 1e-5)
    z1 = (h - mu) * rs * fg_ref[0][None, :] + fb_ref[0][None, :]
    z2 = (inc - mu) * rs * fg_ref[1][None, :] + fb_ref[1][None, :]
    bm = h.shape[0]
    lane = lax.broadcasted_iota(jnp.int32, (bm, 128), 1)
    y = jnp.zeros((bm, 128), F32)
    for d in range(3):
        u = _celu(jnp.dot(z1, w1s_ref[d, :128, :], preferred_element_type=F32)
                  + jnp.dot(z2, w1s_ref[d, 128:, :], preferred_element_type=F32)
                  + b1s_ref[d][None, :])
        yd = jnp.sum(u * w2s_ref[d][None, :], axis=-1, keepdims=True)
        y = jnp.where(lane == d, yd, y)
    o_ref[...] = (y + b2v_ref[...]) * bcm_ref[...]


def _final(h, qpartials, fg, fb, w1s, b1s, w2s, b2v, bcm, bm=1000):
    n = h.shape[0]
    return pl.pallas_call(
        _final_body,
        grid=(n // bm,),
        in_specs=[
            pl.BlockSpec((bm, 128), lambda i: (i, 0)),
            pl.BlockSpec((1, bm, 128), lambda i: (0, i, 0)),
            pl.BlockSpec((1, bm, 128), lambda i: (1, i, 0)),
            pl.BlockSpec((2, 128), lambda i: (0, 0)),
            pl.BlockSpec((2, 128), lambda i: (0, 0)),
            pl.BlockSpec((3, 256, 128), lambda i: (0, 0, 0)),
            pl.BlockSpec((3, 128), lambda i: (0, 0)),
            pl.BlockSpec((3, 128), lambda i: (0, 0)),
            pl.BlockSpec((1, 128), lambda i: (0, 0)),
            pl.BlockSpec((bm, 128), lambda i: (i, 0)),
        ],
        out_specs=pl.BlockSpec((bm, 128), lambda i: (i, 0)),
        out_shape=jax.ShapeDtypeStruct((n, 128), F32),
    )(h, qpartials, qpartials, fg, fb, w1s, b1s, w2s, b2v, bcm)


def _sc_gather(table, idx2):
    nt = table.shape[0] // 2
    nch = idx2.shape[1]
    tps = nch // _NS
    nloop = tps // 2
    rsmall = (nt // _NS) & ~7
    rbig = nt - (_NS - 1) * rsmall
    mesh = plsc.VectorSubcoreMesh(core_axis_name="c", subcore_axis_name="s")

    @functools.partial(
        pl.kernel,
        out_type=jax.ShapeDtypeStruct((2, nch * _CH, 128), F32),
        mesh=mesh,
        scratch_types=[
            pltpu.VMEM((_CH,), jnp.int32),
            pltpu.VMEM((_CH,), jnp.int32),
            pltpu.VMEM((_CH, 128), F32),
            pltpu.VMEM((_CH, 128), F32),
            pltpu.VMEM_SHARED((nt, 128), F32),
            pltpu.SemaphoreType.DMA,
            pltpu.SemaphoreType.DMA,
            pltpu.SemaphoreType.DMA,
            pltpu.SemaphoreType.DMA,
            pltpu.SemaphoreType.DMA,
            pltpu.SemaphoreType.DMA,
        ],
    )
    def k(t_hbm, i_hbm, o_hbm, idx0, idx1, rows0, rows1, tab_sh,
          si0, si1, sg0, sg1, sw0, sw1):
        cid = lax.axis_index("c")
        sid = lax.axis_index("s")
        base = sid * rsmall

        @pl.when(sid < _NS - 1)
        def _():
            pltpu.sync_copy(t_hbm.at[cid, pl.ds(base, rsmall)],
                            tab_sh.at[pl.ds(base, rsmall)])

        @pl.when(sid == _NS - 1)
        def _():
            pltpu.sync_copy(t_hbm.at[cid, pl.ds(base, rbig)],
                            tab_sh.at[pl.ds(base, rbig)])

        def ck(t):
            return t * _NS + sid

        pltpu.async_copy(i_hbm.at[cid, ck(0)], idx0, si0)
        pltpu.async_copy(i_hbm.at[cid, ck(1)], idx1, si1)
        plsc.subcore_barrier()

        def half(g, t, ib, si, rb, sg, sw):
            pltpu.make_async_copy(i_hbm.at[0, 0], ib, si).wait()

            @pl.when(g >= 1)
            def _():
                pltpu.make_async_copy(rb, o_hbm.at[0, pl.ds(0, _CH)],
                                      sw).wait()

            pltpu.async_copy(tab_sh.at[ib], rb, sg)
            pltpu.make_async_copy(tab_sh.at[ib], rb, sg).wait()
            pltpu.async_copy(rb, o_hbm.at[cid, pl.ds(ck(t) * _CH, _CH)], sw)

            @pl.when(t + 2 < tps)
            def _():
                pltpu.async_copy(i_hbm.at[cid, ck(t + 2)], ib, si)

        def body(g, carry):
            half(g, 2 * g, idx0, si0, rows0, sg0, sw0)
            half(g, 2 * g + 1, idx1, si1, rows1, sg1, sw1)
            return carry

        lax.fori_loop(0, nloop, body, 0)
        pltpu.make_async_copy(rows0, o_hbm.at[0, pl.ds(0, _CH)], sw0).wait()
        pltpu.make_async_copy(rows1, o_hbm.at[0, pl.ds(0, _CH)], sw1).wait()

    return k(table.reshape(2, nt, 128), idx2)


def _sc_scatter2(vals0, vals1, idx2, zeros_rows, n_acc):
    nch = idx2.shape[1]
    tps = -(-nch // _NS)
    rsmall = (n_acc // _NS) & ~7
    rbig = n_acc - (_NS - 1) * rsmall
    mesh = plsc.VectorSubcoreMesh(core_axis_name="c", subcore_axis_name="s")

    @functools.partial(
        pl.kernel,
        out_type=jax.ShapeDtypeStruct((2, n_acc, 128), F32),
        mesh=mesh,
        scratch_types=[
            pltpu.VMEM((_CH,), jnp.int32),
            pltpu.VMEM((_CH, 128), F32),
            pltpu.VMEM_SHARED((n_acc, 128), F32),
        ],
    )
    def k(v0_hbm, v1_hbm, i_hbm, z_hbm, o_hbm, idx_v, val_v, acc_sh):
        cid = lax.axis_index("c")
        sid = lax.axis_index("s")
        base = sid * rsmall

        @pl.when(sid < _NS - 1)
        def _():
            pltpu.sync_copy(z_hbm.at[pl.ds(0, rsmall)],
                            acc_sh.at[pl.ds(base, rsmall)])

        @pl.when(sid == _NS - 1)
        def _():
            pltpu.sync_copy(z_hbm.at[pl.ds(0, rbig)],
                            acc_sh.at[pl.ds(base, rbig)])

        plsc.subcore_barrier()

        def body(t, carry):
            ck = sid + _NS * t

            @pl.when(ck < nch)
            def _():
                pltpu.sync_copy(i_hbm.at[cid, ck], idx_v)

                @pl.when(cid == 0)
                def _():
                    pltpu.sync_copy(v0_hbm.at[pl.ds(ck * _CH, _CH)], val_v)

                @pl.when(cid == 1)
                def _():
                    pltpu.sync_copy(v1_hbm.at[pl.ds(ck * _CH, _CH)], val_v)

                pltpu.sync_copy(val_v, acc_sh.at[idx_v], add=True)

            return carry

        lax.fori_loop(0, tps, body, 0)
        plsc.subcore_barrier()

        @pl.when(sid < _NS - 1)
        def _():
            pltpu.sync_copy(acc_sh.at[pl.ds(base, rsmall)],
                            o_hbm.at[cid, pl.ds(base, rsmall)])

        @pl.when(sid == _NS - 1)
        def _():
            pltpu.sync_copy(acc_sh.at[pl.ds(base, rbig)],
                            o_hbm.at[cid, pl.ds(base, rbig)])

    return k(vals0, vals1, idx2, zeros_rows)


def kernel(x, edge_index, edge_attr, bc_disp, bc_rot, params):
    n = x.shape[0]
    e2 = edge_index.shape[1]
    em = e2 // 2

    ei = edge_index.astype(jnp.int32)
    dst = ei[1, :em]
    src = ei[0, :em]
    tps = -(-(em // _CH) // _NS)
    gpad = _NS * tps * _CH - em
    zpad = jnp.zeros((gpad,), jnp.int32)
    gidx = jnp.stack([jnp.concatenate([dst, zpad]),
                      jnp.concatenate([src, zpad])]).reshape(2, -1, _CH)
    sidx = jnp.stack([dst.reshape(-1, _CH), src.reshape(-1, _CH)])
    fidx = jnp.stack([ei[1, :em].reshape(-1, _CH),
                      ei[1, em:].reshape(-1, _CH)])
    rbig = n - (_NS - 1) * ((n // _NS) & ~7)
    zeros_rows = jnp.zeros((rbig, 128), F32)

    ne = params["node_encoder"]
    xpad = jnp.pad(x, ((0, 0), (0, 16 - x.shape[1])))
    w1n = jnp.pad(ne["Ws"][0], ((0, 16 - x.shape[1]), (0, 0)))
    h = _mlp2_ln(xpad, w1n, ne["bs"][0], ne["Ws"][1], ne["bs"][1],
                 ne["ln"][0], ne["ln"][1], bm=1000)

    ee = params["edge_encoder"]
    apad = jnp.pad(edge_attr, ((0, 0), (0, 8 - edge_attr.shape[1])))
    w1e = jnp.pad(ee["Ws"][0], ((0, 8 - edge_attr.shape[1]), (0, 0)))
    e0 = _mlp2_ln(apad, w1e, ee["bs"][0], ee["Ws"][1], ee["bs"][1],
                  ee["ln"][0], ee["ln"][1], bm=1000)
    e_fwd = e0[:em]

    for layer in params["mp_layers"]:
        emlp, nmlp = layer["edge_mlp"], layer["node_mlp"]
        w1 = emlp["Ws"][0]
        w1a, w1b, w1c = w1[:128], w1[128:256], w1[256:]
        a = _edge_pre(e_fwd, w1a)
        table = _tables(h, w1b, w1c)
        g = _sc_gather(table, gidx)
        msg, e_fwd = _edge_post(a, g, e_fwd, emlp["bs"][0], emlp["Ws"][1],
                                emlp["bs"][1], emlp["ln"][0], emlp["ln"][1])
        partials = _sc_scatter2(msg, msg, sidx, zeros_rows, n)
        v1 = nmlp["Ws"][0]
        h = _node_update(h, partials, v1[:128], v1[128:], nmlp["bs"][0],
                         nmlp["Ws"][1], nmlp["bs"][1], nmlp["ln"][0],
                         nmlp["ln"][1])

    e_bwd = _ebwd(e0, e_fwd)
    qpartials = _sc_scatter2(e_fwd, e_bwd, fidx, zeros_rows, n)

    fg, fb = params["final_norm"]
    dux, duz, dth = (params["decoder_ux"], params["decoder_uz"],
                     params["decoder_th"])
    w1s = jnp.stack([dux["Ws"][0], duz["Ws"][0], dth["Ws"][0]])
    b1s = jnp.stack([dux["bs"][0], duz["bs"][0], dth["bs"][0]])
    w2s = jnp.stack([dux["Ws"][1][:, 0], duz["Ws"][1][:, 0], dth["Ws"][1][:, 0]])
    b2v = jnp.pad(jnp.stack([dux["bs"][1][0], duz["bs"][1][0],
                             dth["bs"][1][0]]).reshape(1, 3),
                  ((0, 0), (0, 125)))
    bcm = jnp.pad(jnp.concatenate([1.0 - bc_disp, 1.0 - bc_disp,
                                   1.0 - bc_rot], axis=1),
                  ((0, 0), (0, 125)))
    ypad = _final(h, qpartials, fg.reshape(2, 128), fb.reshape(2, 128),
                  w1s, b1s, w2s, b2v, bcm)
    return ypad[:, :3]


# speedup vs baseline: 1.1007x; 1.1007x over previous
import functools

import jax
import jax.numpy as jnp
from jax import lax
from jax.experimental import pallas as pl
from jax.experimental.pallas import tpu as pltpu
from jax.experimental.pallas import tpu_sc as plsc

F32 = jnp.float32
_NC, _NS = 2, 16
_NW = _NC * _NS
_CH = 128


def _celu(u):
    return jnp.where(u > 0, u, jnp.exp(jnp.minimum(u, 0.0)) - 1.0)


def _ln(y, g, b):
    mu = jnp.mean(y, axis=-1, keepdims=True)
    var = jnp.mean((y - mu) ** 2, axis=-1, keepdims=True)
    return (y - mu) * lax.rsqrt(var + 1e-5) * g + b


def _mlp2_ln_body(x_ref, w1_ref, b1_ref, w2_ref, b2_ref, g_ref, be_ref, o_ref):
    u = _celu(jnp.dot(x_ref[...], w1_ref[...], preferred_element_type=F32)
              + b1_ref[...])
    y = jnp.dot(u, w2_ref[...], preferred_element_type=F32) + b2_ref[...]
    o_ref[...] = _ln(y, g_ref[...], be_ref[...])


def _mlp2_ln(x, w1, b1, w2, b2, g, be, bm):
    n, kdim = x.shape
    grid = n // bm
    return pl.pallas_call(
        _mlp2_ln_body,
        grid=(grid,),
        in_specs=[
            pl.BlockSpec((bm, kdim), lambda i: (i, 0)),
            pl.BlockSpec((kdim, 128), lambda i: (0, 0)),
            pl.BlockSpec((1, 128), lambda i: (0, 0)),
            pl.BlockSpec((128, 128), lambda i: (0, 0)),
            pl.BlockSpec((1, 128), lambda i: (0, 0)),
            pl.BlockSpec((1, 128), lambda i: (0, 0)),
            pl.BlockSpec((1, 128), lambda i: (0, 0)),
        ],
        out_specs=pl.BlockSpec((bm, 128), lambda i: (i, 0)),
        out_shape=jax.ShapeDtypeStruct((n, 128), F32),
    )(x, w1, b1.reshape(1, 128), w2, b2.reshape(1, 128),
      g.reshape(1, 128), be.reshape(1, 128))


def _matmul_body(x_ref, w_ref, o_ref):
    o_ref[...] = jnp.dot(x_ref[...], w_ref[...], preferred_element_type=F32)


def _edge_pre(e_fwd, w1a, bm=1000):
    n = e_fwd.shape[0]
    return pl.pallas_call(
        _matmul_body,
        grid=(n // bm,),
        in_specs=[
            pl.BlockSpec((bm, 128), lambda i: (i, 0)),
            pl.BlockSpec((128, 128), lambda i: (0, 0)),
        ],
        out_specs=pl.BlockSpec((bm, 128), lambda i: (i, 0)),
        out_shape=jax.ShapeDtypeStruct((n, 128), F32),
    )(e_fwd, w1a)


def _tables_body(h_ref, w_ref, o_ref):
    o_ref[...] = jnp.dot(h_ref[...], w_ref[0], preferred_element_type=F32)


def _tables(h, w1b, w1c, bm=1000):
    n = h.shape[0]
    nb = n // bm
    wbc = jnp.stack([w1b, w1c])
    return pl.pallas_call(
        _tables_body,
        grid=(2 * nb,),
        in_specs=[
            pl.BlockSpec((bm, 128), lambda i: (i % nb, 0)),
            pl.BlockSpec((1, 128, 128), lambda i: (i // nb, 0, 0)),
        ],
        out_specs=pl.BlockSpec((bm, 128), lambda i: (i, 0)),
        out_shape=jax.ShapeDtypeStruct((2 * n, 128), F32),
    )(h, wbc)


def _edge_post_body(a_ref, gp_ref, gq_ref, e_ref, b1_ref, w2_ref, b2_ref,
                    g_ref, be_ref, msg_ref, enew_ref):
    u = _celu(a_ref[...] + gp_ref[0] + gq_ref[0] + b1_ref[...])
    m = _ln(jnp.dot(u, w2_ref[...], preferred_element_type=F32) + b2_ref[...],
            g_ref[...], be_ref[...])
    msg_ref[...] = m
    enew_ref[...] = e_ref[...] + m


def _edge_post(a, gfull, e_fwd, b1, w2, b2, g, be, bm=1000):
    n = a.shape[0]
    nb = n // bm
    return pl.pallas_call(
        _edge_post_body,
        grid=(nb,),
        in_specs=[
            pl.BlockSpec((bm, 128), lambda i: (i, 0)),
            pl.BlockSpec((1, bm, 128), lambda i: (0, i, 0)),
            pl.BlockSpec((1, bm, 128), lambda i: (1, i, 0)),
            pl.BlockSpec((bm, 128), lambda i: (i, 0)),
            pl.BlockSpec((1, 128), lambda i: (0, 0)),
            pl.BlockSpec((128, 128), lambda i: (0, 0)),
            pl.BlockSpec((1, 128), lambda i: (0, 0)),
            pl.BlockSpec((1, 128), lambda i: (0, 0)),
            pl.BlockSpec((1, 128), lambda i: (0, 0)),
        ],
        out_specs=[
            pl.BlockSpec((bm, 128), lambda i: (i, 0)),
            pl.BlockSpec((bm, 128), lambda i: (i, 0)),
        ],
        out_shape=[
            jax.ShapeDtypeStruct((n, 128), F32),
            jax.ShapeDtypeStruct((n, 128), F32),
        ],
    )(a, gfull, gfull, e_fwd, b1.reshape(1, 128), w2, b2.reshape(1, 128),
      g.reshape(1, 128), be.reshape(1, 128))


def _node_body(h_ref, p0_ref, p1_ref, v1a_ref, v1b_ref, c1_ref, v2_ref,
               c2_ref, g_ref, be_ref, o_ref):
    agg = p0_ref[0] - p1_ref[0]
    u = _celu(jnp.dot(h_ref[...], v1a_ref[...], preferred_element_type=F32)
              + jnp.dot(agg, v1b_ref[...], preferred_element_type=F32)
              + c1_ref[...])
    y = _ln(jnp.dot(u, v2_ref[...], preferred_element_type=F32) + c2_ref[...],
            g_ref[...], be_ref[...])
    o_ref[...] = h_ref[...] + y


def _node_update(h, partials, v1a, v1b, c1, v2, c2, g, be, bm=1000):
    n = h.shape[0]
    return pl.pallas_call(
        _node_body,
        grid=(n // bm,),
        in_specs=[
            pl.BlockSpec((bm, 128), lambda i: (i, 0)),
            pl.BlockSpec((1, bm, 128), lambda i: (0, i, 0)),
            pl.BlockSpec((1, bm, 128), lambda i: (1, i, 0)),
            pl.BlockSpec((128, 128), lambda i: (0, 0)),
            pl.BlockSpec((128, 128), lambda i: (0, 0)),
            pl.BlockSpec((1, 128), lambda i: (0, 0)),
            pl.BlockSpec((128, 128), lambda i: (0, 0)),
            pl.BlockSpec((1, 128), lambda i: (0, 0)),
            pl.BlockSpec((1, 128), lambda i: (0, 0)),
            pl.BlockSpec((1, 128), lambda i: (0, 0)),
        ],
        out_specs=pl.BlockSpec((bm, 128), lambda i: (i, 0)),
        out_shape=jax.ShapeDtypeStruct((n, 128), F32),
    )(h, partials, partials, v1a, v1b, c1.reshape(1, 128), v2,
      c2.reshape(1, 128), g.reshape(1, 128), be.reshape(1, 128))


def _ebwd_body(e0f_ref, e0b_ref, ef_ref, o_ref):
    o_ref[...] = e0b_ref[...] - (ef_ref[...] - e0f_ref[...])


def _ebwd(e0, ef, bm=1000):
    n = ef.shape[0]
    nb = n // bm
    return pl.pallas_call(
        _ebwd_body,
        grid=(nb,),
        in_specs=[
            pl.BlockSpec((bm, 128), lambda i: (i, 0)),
            pl.BlockSpec((bm, 128), lambda i: (i + nb, 0)),
            pl.BlockSpec((bm, 128), lambda i: (i, 0)),
        ],
        out_specs=pl.BlockSpec((bm, 128), lambda i: (i, 0)),
        out_shape=jax.ShapeDtypeStruct((n, 128), F32),
    )(e0, e0, ef)


def _final_body(h_ref, q0_ref, q1_ref, fg_ref, fb_ref, w1s_ref, b1s_ref,
                w2s_ref, b2v_ref, bcm_ref, o_ref):
    h = h_ref[...]
    inc = q0_ref[0] + q1_ref[0]
    s = jnp.sum(h, axis=-1, keepdims=True) + jnp.sum(inc, axis=-1, keepdims=True)
    mu = s / 256.0
    v = (jnp.sum((h - mu) ** 2, axis=-1, keepdims=True)
         + jnp.sum((inc - mu) ** 2, axis=-1, keepdims=True)) / 256.0
    rs = lax.rsqrt(v + 1e-5)
    z1 = (h - mu) * rs * fg_ref[0][None, :] + fb_ref[0][None, :]
    z2 = (inc - mu) * rs * fg_ref[1][None, :] + fb_ref[1][None, :]
    bm = h.shape[0]
    lane = lax.broadcasted_iota(jnp.int32, (bm, 128), 1)
    y = jnp.zeros((bm, 128), F32)
    for d in range(3):
        u = _celu(jnp.dot(z1, w1s_ref[d, :128, :], preferred_element_type=F32)
                  + jnp.dot(z2, w1s_ref[d, 128:, :], preferred_element_type=F32)
                  + b1s_ref[d][None, :])
        yd = jnp.sum(u * w2s_ref[d][None, :], axis=-1, keepdims=True)
        y = jnp.where(lane == d, yd, y)
    o_ref[...] = (y + b2v_ref[...]) * bcm_ref[...]


def _final(h, qpartials, fg, fb, w1s, b1s, w2s, b2v, bcm, bm=1000):
    n = h.shape[0]
    return pl.pallas_call(
        _final_body,
        grid=(n // bm,),
        in_specs=[
            pl.BlockSpec((bm, 128), lambda i: (i, 0)),
            pl.BlockSpec((1, bm, 128), lambda i: (0, i, 0)),
            pl.BlockSpec((1, bm, 128), lambda i: (1, i, 0)),
            pl.BlockSpec((2, 128), lambda i: (0, 0)),
            pl.BlockSpec((2, 128), lambda i: (0, 0)),
            pl.BlockSpec((3, 256, 128), lambda i: (0, 0, 0)),
            pl.BlockSpec((3, 128), lambda i: (0, 0)),
            pl.BlockSpec((3, 128), lambda i: (0, 0)),
            pl.BlockSpec((1, 128), lambda i: (0, 0)),
            pl.BlockSpec((bm, 128), lambda i: (i, 0)),
        ],
        out_specs=pl.BlockSpec((bm, 128), lambda i: (i, 0)),
        out_shape=jax.ShapeDtypeStruct((n, 128), F32),
    )(h, qpartials, qpartials, fg, fb, w1s, b1s, w2s, b2v, bcm)


def _sc_gather(table, idx2):
    nt = table.shape[0] // 2
    nch = idx2.shape[1]
    tps = nch // _NS
    nloop = tps // 2
    rsmall = (nt // _NS) & ~7
    rbig = nt - (_NS - 1) * rsmall
    mesh = plsc.VectorSubcoreMesh(core_axis_name="c", subcore_axis_name="s")

    @functools.partial(
        pl.kernel,
        out_type=jax.ShapeDtypeStruct((2, nch * _CH, 128), F32),
        mesh=mesh,
        scratch_types=[
            pltpu.VMEM((_CH,), jnp.int32),
            pltpu.VMEM((_CH,), jnp.int32),
            pltpu.VMEM((_CH, 128), F32),
            pltpu.VMEM((_CH, 128), F32),
            pltpu.VMEM_SHARED((nt, 128), F32),
            pltpu.SemaphoreType.DMA,
            pltpu.SemaphoreType.DMA,
            pltpu.SemaphoreType.DMA,
            pltpu.SemaphoreType.DMA,
            pltpu.SemaphoreType.DMA,
            pltpu.SemaphoreType.DMA,
        ],
    )
    def k(t_hbm, i_hbm, o_hbm, idx0, idx1, rows0, rows1, tab_sh,
          si0, si1, sg0, sg1, sw0, sw1):
        cid = lax.axis_index("c")
        sid = lax.axis_index("s")
        base = sid * rsmall

        @pl.when(sid < _NS - 1)
        def _():
            pltpu.sync_copy(t_hbm.at[cid, pl.ds(base, rsmall)],
                            tab_sh.at[pl.ds(base, rsmall)])

        @pl.when(sid == _NS - 1)
        def _():
            pltpu.sync_copy(t_hbm.at[cid, pl.ds(base, rbig)],
                            tab_sh.at[pl.ds(base, rbig)])

        def ck(t):
            return t * _NS + sid

        pltpu.async_copy(i_hbm.at[cid, ck(0)], idx0, si0)
        pltpu.async_copy(i_hbm.at[cid, ck(1)], idx1, si1)
        plsc.subcore_barrier()

        def half(g, t, ib, si, rb, sg, sw):
            pltpu.make_async_copy(i_hbm.at[0, 0], ib, si).wait()

            @pl.when(g >= 1)
            def _():
                pltpu.make_async_copy(rb, o_hbm.at[0, pl.ds(0, _CH)],
                                      sw).wait()

            pltpu.async_copy(tab_sh.at[ib], rb, sg)
            pltpu.make_async_copy(tab_sh.at[ib], rb, sg).wait()
            pltpu.async_copy(rb, o_hbm.at[cid, pl.ds(ck(t) * _CH, _CH)], sw)

            @pl.when(t + 2 < tps)
            def _():
                pltpu.async_copy(i_hbm.at[cid, ck(t + 2)], ib, si)

        def body(g, carry):
            half(g, 2 * g, idx0, si0, rows0, sg0, sw0)
            half(g, 2 * g + 1, idx1, si1, rows1, sg1, sw1)
            return carry

        lax.fori_loop(0, nloop, body, 0)
        pltpu.make_async_copy(rows0, o_hbm.at[0, pl.ds(0, _CH)], sw0).wait()
        pltpu.make_async_copy(rows1, o_hbm.at[0, pl.ds(0, _CH)], sw1).wait()

    return k(table.reshape(2, nt, 128), idx2)


def _sc_scatter2(vals0, vals1, idx2, zeros_rows, n_acc, n_rows):
    nch = idx2.shape[1]
    tps = -(-nch // _NS)
    nloop = tps // 2
    maxck = n_rows // _CH - 1
    rsmall = (n_acc // _NS) & ~7
    rbig = n_acc - (_NS - 1) * rsmall
    mesh = plsc.VectorSubcoreMesh(core_axis_name="c", subcore_axis_name="s")

    @functools.partial(
        pl.kernel,
        out_type=jax.ShapeDtypeStruct((2, n_acc, 128), F32),
        mesh=mesh,
        scratch_types=[
            pltpu.VMEM((_CH,), jnp.int32),
            pltpu.VMEM((_CH,), jnp.int32),
            pltpu.VMEM((_CH, 128), F32),
            pltpu.VMEM((_CH, 128), F32),
            pltpu.VMEM_SHARED((n_acc, 128), F32),
            pltpu.SemaphoreType.DMA,
            pltpu.SemaphoreType.DMA,
            pltpu.SemaphoreType.DMA,
            pltpu.SemaphoreType.DMA,
            pltpu.SemaphoreType.DMA,
            pltpu.SemaphoreType.DMA,
        ],
    )
    def k(v0_hbm, v1_hbm, i_hbm, z_hbm, o_hbm, idx0, idx1, val0, val1,
          acc_sh, si0, si1, sv0, sv1, sc0, sc1):
        cid = lax.axis_index("c")
        sid = lax.axis_index("s")
        base = sid * rsmall

        @pl.when(sid < _NS - 1)
        def _():
            pltpu.sync_copy(z_hbm.at[pl.ds(0, rsmall)],
                            acc_sh.at[pl.ds(base, rsmall)])

        @pl.when(sid == _NS - 1)
        def _():
            pltpu.sync_copy(z_hbm.at[pl.ds(0, rbig)],
                            acc_sh.at[pl.ds(base, rbig)])

        def ck(t):
            return t * _NS + sid

        def vrow(t):
            return jnp.minimum(ck(t), maxck) * _CH

        def load(t, ib, si, vb, sv):
            pltpu.async_copy(i_hbm.at[cid, ck(t)], ib, si)

            @pl.when(cid == 0)
            def _():
                pltpu.async_copy(v0_hbm.at[pl.ds(vrow(t), _CH)], vb, sv)

            @pl.when(cid == 1)
            def _():
                pltpu.async_copy(v1_hbm.at[pl.ds(vrow(t), _CH)], vb, sv)

        def wait_load(ib, si, vb, sv):
            pltpu.make_async_copy(i_hbm.at[0, 0], ib, si).wait()
            pltpu.make_async_copy(v0_hbm.at[pl.ds(0, _CH)], vb, sv).wait()

        def wait_scat(ib, vb, sc):
            pltpu.make_async_copy(vb, acc_sh.at[ib], sc).wait()

        load(0, idx0, si0, val0, sv0)
        load(1, idx1, si1, val1, sv1)
        plsc.subcore_barrier()

        def body(g, carry):
            t0 = 2 * g
            wait_load(idx0, si0, val0, sv0)
            pltpu.async_copy(val0, acc_sh.at[idx0], sc0, add=True)
            wait_load(idx1, si1, val1, sv1)
            pltpu.async_copy(val1, acc_sh.at[idx1], sc1, add=True)
            wait_scat(idx0, val0, sc0)

            @pl.when(t0 + 2 < tps)
            def _():
                load(t0 + 2, idx0, si0, val0, sv0)

            wait_scat(idx1, val1, sc1)

            @pl.when(t0 + 3 < tps)
            def _():
                load(t0 + 3, idx1, si1, val1, sv1)

            return carry

        lax.fori_loop(0, nloop, body, 0)
        plsc.subcore_barrier()

        @pl.when(sid < _NS - 1)
        def _():
            pltpu.sync_copy(acc_sh.at[pl.ds(base, rsmall)],
                            o_hbm.at[cid, pl.ds(base, rsmall)])

        @pl.when(sid == _NS - 1)
        def _():
            pltpu.sync_copy(acc_sh.at[pl.ds(base, rbig)],
                            o_hbm.at[cid, pl.ds(base, rbig)])

    return k(vals0, vals1, idx2, zeros_rows)


def kernel(x, edge_index, edge_attr, bc_disp, bc_rot, params):
    n = x.shape[0]
    e2 = edge_index.shape[1]
    em = e2 // 2

    ei = edge_index.astype(jnp.int32)
    dst = ei[1, :em]
    src = ei[0, :em]
    tps = -(-(em // _CH) // _NS)
    gpad = _NS * tps * _CH - em
    zpad = jnp.zeros((gpad,), jnp.int32)
    gidx = jnp.stack([jnp.concatenate([dst, zpad]),
                      jnp.concatenate([src, zpad])]).reshape(2, -1, _CH)
    n_acc = n + _NS
    dump = jnp.full((gpad,), n_acc - 1, jnp.int32)
    sidx = jnp.stack([jnp.concatenate([dst, dump]),
                      jnp.concatenate([src, dump])]).reshape(2, -1, _CH)
    fidx = jnp.stack([jnp.concatenate([ei[1, :em], dump]),
                      jnp.concatenate([ei[1, em:], dump])]).reshape(2, -1, _CH)
    rbig = n_acc - (_NS - 1) * ((n_acc // _NS) & ~7)
    zeros_rows = jnp.zeros((rbig, 128), F32)

    ne = params["node_encoder"]
    xpad = jnp.pad(x, ((0, 0), (0, 16 - x.shape[1])))
    w1n = jnp.pad(ne["Ws"][0], ((0, 16 - x.shape[1]), (0, 0)))
    h = _mlp2_ln(xpad, w1n, ne["bs"][0], ne["Ws"][1], ne["bs"][1],
                 ne["ln"][0], ne["ln"][1], bm=1000)

    ee = params["edge_encoder"]
    apad = jnp.pad(edge_attr, ((0, 0), (0, 8 - edge_attr.shape[1])))
    w1e = jnp.pad(ee["Ws"][0], ((0, 8 - edge_attr.shape[1]), (0, 0)))
    e0 = _mlp2_ln(apad, w1e, ee["bs"][0], ee["Ws"][1], ee["bs"][1],
                  ee["ln"][0], ee["ln"][1], bm=1000)
    e_fwd = e0[:em]

    for layer in params["mp_layers"]:
        emlp, nmlp = layer["edge_mlp"], layer["node_mlp"]
        w1 = emlp["Ws"][0]
        w1a, w1b, w1c = w1[:128], w1[128:256], w1[256:]
        a = _edge_pre(e_fwd, w1a)
        table = _tables(h, w1b, w1c)
        g = _sc_gather(table, gidx)
        msg, e_fwd = _edge_post(a, g, e_fwd, emlp["bs"][0], emlp["Ws"][1],
                                emlp["bs"][1], emlp["ln"][0], emlp["ln"][1])
        partials = _sc_scatter2(msg, msg, sidx, zeros_rows, n_acc, em)
        v1 = nmlp["Ws"][0]
        h = _node_update(h, partials, v1[:128], v1[128:], nmlp["bs"][0],
                         nmlp["Ws"][1], nmlp["bs"][1], nmlp["ln"][0],
                         nmlp["ln"][1])

    e_bwd = _ebwd(e0, e_fwd)
    qpartials = _sc_scatter2(e_fwd, e_bwd, fidx, zeros_rows, n_acc, em)

    fg, fb = params["final_norm"]
    dux, duz, dth = (params["decoder_ux"], params["decoder_uz"],
                     params["decoder_th"])
    w1s = jnp.stack([dux["Ws"][0], duz["Ws"][0], dth["Ws"][0]])
    b1s = jnp.stack([dux["bs"][0], duz["bs"][0], dth["bs"][0]])
    w2s = jnp.stack([dux["Ws"][1][:, 0], duz["Ws"][1][:, 0], dth["Ws"][1][:, 0]])
    b2v = jnp.pad(jnp.stack([dux["bs"][1][0], duz["bs"][1][0],
                             dth["bs"][1][0]]).reshape(1, 3),
                  ((0, 0), (0, 125)))
    bcm = jnp.pad(jnp.concatenate([1.0 - bc_disp, 1.0 - bc_disp,
                                   1.0 - bc_rot], axis=1),
                  ((0, 0), (0, 125)))
    ypad = _final(h, qpartials, fg.reshape(2, 128), fb.reshape(2, 128),
                  w1s, b1s, w2s, b2v, bcm)
    return ypad[:, :3]
